# baseline (speedup 1.0000x reference)
import functools

import jax
import jax.numpy as jnp
from jax import lax
from jax.experimental import pallas as pl
from jax.experimental.pallas import tpu as pltpu
from jax.experimental.pallas import tpu_sc as plsc

_NC = 2
_NS = 16
_LANES = 16
_SW = 32


def _transpose_sum(vecs, lane_iota):
    k = 1
    while len(vecs) > 1 or k <= _LANES // 2:
        mask = (lane_iota & k) != 0
        idx = lane_iota ^ k
        nxt = []
        for i in range(0, len(vecs), 2):
            a = vecs[i]
            b = vecs[i + 1] if i + 1 < len(vecs) else a
            first = jnp.where(mask, b, a)
            second = jnp.take_along_axis(jnp.where(mask, a, b), idx, axis=0)
            nxt.append(first + second)
        vecs = nxt
        k *= 2
    return vecs[0]


def _sc_scores(i_gidx, on_gidx, i_emb, o_emb, *, B, T, D, chunk):
    NW = _NC * _NS
    bpw = B // NW
    C = chunk
    nch = bpw // C
    KD = D // _LANES
    G = NW * nch

    mesh = plsc.VectorSubcoreMesh(core_axis_name="c", subcore_axis_name="s")

    @functools.partial(
        pl.kernel,
        out_type=jax.ShapeDtypeStruct((G, C, _SW), jnp.float32),
        mesh=mesh,
        compiler_params=pltpu.CompilerParams(use_tc_tiling_on_sc=False),
        scratch_types=[
            pltpu.VMEM((nch, C), jnp.int32),
            pltpu.VMEM((T * nch, C), jnp.int32),
            pltpu.VMEM((C, D), jnp.float32),
            pltpu.VMEM((T, C, D), jnp.float32),
            pltpu.VMEM((nch, C, _SW), jnp.float32),
            pltpu.SemaphoreType.DMA,
        ],
    )
    def scores_kernel(i_gidx_hbm, on_gidx_hbm, i_tab, o_tab,
                      out_hbm, i_gidx_v, on_gidx_v,
                      i_rows_v, on_rows_v, scores_v, sem):
        wid = lax.axis_index("s") * _NC + lax.axis_index("c")
        lane_iota = lax.iota(jnp.int32, _LANES)

        pltpu.sync_copy(i_gidx_hbm.at[wid], i_gidx_v)
        pltpu.sync_copy(on_gidx_hbm.at[wid], on_gidx_v)

        def chunk_body(ci, carry):
            cps = [pltpu.async_copy(
                i_tab.at[i_gidx_v.at[ci]], i_rows_v, sem)]
            for t in range(T):
                cps.append(pltpu.async_copy(
                    o_tab.at[on_gidx_v.at[t * nch + ci]],
                    on_rows_v.at[t], sem))
            for cp in cps:
                cp.wait()

            def elem_body(e, carry2):
                iv = [i_rows_v[e, pl.ds(kk * _LANES, _LANES)]
                      for kk in range(KD)]
                niv = [-v for v in iv]
                accs = []
                for t in range(T):
                    src = iv if t == 0 else niv
                    acc = src[0] * on_rows_v[t, e, pl.ds(0, _LANES)]
                    for kk in range(1, KD):
                        acc = acc + src[kk] * on_rows_v[
                            t, e, pl.ds(kk * _LANES, _LANES)]
                    accs.append(acc)
                sA = _transpose_sum(accs[:_LANES], lane_iota)
                sB = _transpose_sum(accs[_LANES:], lane_iota)
                scores_v[ci, e, pl.ds(0, _LANES)] = sA
                scores_v[ci, e, pl.ds(_LANES, _LANES)] = sB
                return carry2

            lax.fori_loop(0, C, elem_body, 0)
            return carry

        lax.fori_loop(0, nch, chunk_body, 0)
        pltpu.sync_copy(scores_v, out_hbm.at[pl.ds(wid * nch, nch)])

    return scores_kernel(i_gidx, on_gidx, i_emb, o_emb)


def _make_loss_body(T):
    def _loss_body(s_ref, o_ref):
        x = s_ref[...]
        o_ref[...] = jnp.sum(jax.nn.log_sigmoid(x[:, :, :T]), keepdims=True)
    return _loss_body


def kernel(i_words, o_words, n_words, i_emb, o_emb):
    B, S = i_words.shape
    T = 1 + n_words.shape[1]
    V, D = i_emb.shape
    NW = _NC * _NS
    C = 32
    bpw = B // NW
    nch = bpw // C
    i_gidx = i_words.reshape(NW, nch, C)
    on_idx = jnp.concatenate([o_words, n_words], axis=1)
    on_bc = on_idx.reshape(NW, nch, C, T)
    on_tm = jnp.transpose(on_bc, (0, 3, 1, 2))
    on_gidx = on_tm.reshape(NW, T * nch, C)
    scores = _sc_scores(i_gidx, on_gidx, i_emb, o_emb,
                        B=B, T=T, D=D, chunk=C)
    total = pl.pallas_call(
        _make_loss_body(T),
        out_shape=jax.ShapeDtypeStruct((1, 1, 1), jnp.float32),
    )(scores)
    return -total[0, 0, 0] / (B * S)


# speedup vs baseline: 1.0164x; 1.0164x over previous
import functools

import jax
import jax.numpy as jnp
from jax import lax
from jax.experimental import pallas as pl
from jax.experimental.pallas import tpu as pltpu
from jax.experimental.pallas import tpu_sc as plsc

_NC = 2
_NS = 16
_LANES = 16
_SW = 32


def _transpose_sum(vecs, lane_iota):
    k = 1
    while len(vecs) > 1 or k <= _LANES // 2:
        mask = (lane_iota & k) != 0
        idx = lane_iota ^ k
        nxt = []
        for i in range(0, len(vecs), 2):
            a = vecs[i]
            b = vecs[i + 1] if i + 1 < len(vecs) else a
            first = jnp.where(mask, b, a)
            second = jnp.take_along_axis(jnp.where(mask, a, b), idx, axis=0)
            nxt.append(first + second)
        vecs = nxt
        k *= 2
    return vecs[0]


def _sc_scores(i_gidx, on_gidx, i_emb, o_emb, *, B, T, D, chunk):
    NW = _NC * _NS
    bpw = B // NW
    C = chunk
    nch = bpw // C
    KD = D // _LANES
    G = NW * nch

    mesh = plsc.VectorSubcoreMesh(core_axis_name="c", subcore_axis_name="s")

    @functools.partial(
        pl.kernel,
        out_type=jax.ShapeDtypeStruct((G, C, _SW), jnp.float32),
        mesh=mesh,
        compiler_params=pltpu.CompilerParams(use_tc_tiling_on_sc=False),
        scratch_types=[
            pltpu.VMEM((2, C), jnp.int32),
            pltpu.VMEM((2, T, C), jnp.int32),
            pltpu.VMEM((2, C, D), jnp.float32),
            pltpu.VMEM((2, T, C, D), jnp.float32),
            pltpu.VMEM((nch, C, _SW), jnp.float32),
            pltpu.SemaphoreType.DMA,
            pltpu.SemaphoreType.DMA,
        ],
    )
    def scores_kernel(i_gidx_hbm, on_gidx_hbm, i_tab, o_tab,
                      out_hbm, i_idx_v, on_idx_v,
                      i_rows_v, on_rows_v, scores_v, sem0, sem1):
        wid = lax.axis_index("s") * _NC + lax.axis_index("c")
        lane_iota = lax.iota(jnp.int32, _LANES)
        sems = (sem0, sem1)

        def issue(ci, b):
            pltpu.sync_copy(i_gidx_hbm.at[wid, ci], i_idx_v.at[b])
            pltpu.sync_copy(on_gidx_hbm.at[wid, ci], on_idx_v.at[b])
            pltpu.async_copy(i_tab.at[i_idx_v.at[b]], i_rows_v.at[b],
                             sems[b])
            for t in range(T):
                pltpu.async_copy(o_tab.at[on_idx_v.at[b, t]],
                                 on_rows_v.at[b, t], sems[b])

        def drain(b):
            pltpu.make_async_copy(i_tab.at[i_idx_v.at[b]], i_rows_v.at[b],
                                  sems[b]).wait()
            for t in range(T):
                pltpu.make_async_copy(o_tab.at[on_idx_v.at[b, t]],
                                      on_rows_v.at[b, t], sems[b]).wait()

        def compute(ci, b):
            def elem_body(e, carry2):
                iv = [i_rows_v[b, e, pl.ds(kk * _LANES, _LANES)]
                      for kk in range(KD)]
                niv = [-v for v in iv]
                accs = []
                for t in range(T):
                    src = iv if t == 0 else niv
                    acc = src[0] * on_rows_v[b, t, e, pl.ds(0, _LANES)]
                    for kk in range(1, KD):
                        acc = acc + src[kk] * on_rows_v[
                            b, t, e, pl.ds(kk * _LANES, _LANES)]
                    accs.append(acc)
                sA = _transpose_sum(accs[:_LANES], lane_iota)
                sB = _transpose_sum(accs[_LANES:], lane_iota)
                scores_v[ci, e, pl.ds(0, _LANES)] = sA
                scores_v[ci, e, pl.ds(_LANES, _LANES)] = sB
                return carry2

            lax.fori_loop(0, C, elem_body, 0)

        issue(0, 0)

        def pair_body(g, carry):
            ci0 = 2 * g
            issue(ci0 + 1, 1)
            drain(0)
            compute(ci0, 0)

            @pl.when(ci0 + 2 < nch)
            def _():
                issue(ci0 + 2, 0)

            drain(1)
            compute(ci0 + 1, 1)
            return carry

        lax.fori_loop(0, nch // 2, pair_body, 0)
        pltpu.sync_copy(scores_v, out_hbm.at[pl.ds(wid * nch, nch)])

    return scores_kernel(i_gidx, on_gidx, i_emb, o_emb)


def _make_loss_body(T):
    def _loss_body(s_ref, o_ref):
        x = s_ref[...]
        o_ref[...] = jnp.sum(jax.nn.log_sigmoid(x[:, :, :T]), keepdims=True)
    return _loss_body


def kernel(i_words, o_words, n_words, i_emb, o_emb):
    B, S = i_words.shape
    T = 1 + n_words.shape[1]
    V, D = i_emb.shape
    NW = _NC * _NS
    C = 32
    bpw = B // NW
    nch = bpw // C
    i_gidx = i_words.reshape(NW, nch, C)
    on_idx = jnp.concatenate([o_words, n_words], axis=1)
    on_bc = on_idx.reshape(NW, nch, C, T)
    on_gidx = jnp.transpose(on_bc, (0, 1, 3, 2))
    scores = _sc_scores(i_gidx, on_gidx, i_emb, o_emb,
                        B=B, T=T, D=D, chunk=C)
    total = pl.pallas_call(
        _make_loss_body(T),
        out_shape=jax.ShapeDtypeStruct((1, 1, 1), jnp.float32),
    )(scores)
    return -total[0, 0, 0] / (B * S)
